# Initial kernel scaffold; baseline (speedup 1.0000x reference)
import jax
import jax.numpy as jnp
from jax.experimental import pallas as pl


def kernel(x, gate_w, w_gate, w_up, w_down):
    raise NotImplementedError("write your pallas kernel here")


# speedup vs baseline: 2.7787x; 2.7787x over previous
import functools

import jax
import jax.numpy as jnp
from jax import lax
from jax.experimental import pallas as pl
from jax.experimental.pallas import tpu as pltpu
from jax.experimental.pallas import tpu_sc as plsc

_H = 768
_FF = 2048
_E = 8
_S = 2048
_T = 256
_NB = _S // _T + _E
_PS = _NB * _T
_NW = 32
_CHUNK = _S // _NW


def _plan_body(x_ref, gw_ref, pos_ref, be_ref, bv_ref):
    x = x_ref[...]
    gw = gw_ref[...]
    logits = lax.dot_general(
        x.astype(jnp.bfloat16), gw.astype(jnp.bfloat16),
        (((1,), (1,)), ((), ())),
        preferred_element_type=jnp.float32)
    m = jnp.max(logits, axis=1, keepdims=True)
    col = lax.broadcasted_iota(jnp.int32, (_S, _E), 1)
    cand = jnp.where(logits == m, col, _E)
    eid = jnp.min(cand, axis=1, keepdims=True)
    onehot = (col == eid).astype(jnp.float32)

    tri = (lax.broadcasted_iota(jnp.int32, (_T, _T), 0)
           >= lax.broadcasted_iota(jnp.int32, (_T, _T), 1)).astype(jnp.float32)
    chunks = []
    run = jnp.zeros((1, _E), jnp.float32)
    for c in range(_S // _T):
        oh_c = onehot[c * _T:(c + 1) * _T, :]
        w_c = jnp.dot(tri, oh_c, preferred_element_type=jnp.float32)
        chunks.append(w_c + run)
        run = run + w_c[_T - 1:_T, :]
    rank_incl = jnp.concatenate(chunks, axis=0)
    counts = run

    pc = jnp.ceil(counts / _T) * _T
    erow = lax.broadcasted_iota(jnp.int32, (_E, _E), 0)
    ecol = lax.broadcasted_iota(jnp.int32, (_E, _E), 1)
    upper = (erow < ecol).astype(jnp.float32)
    astart = jnp.dot(pc, upper, preferred_element_type=jnp.float32)

    rank_tok = jnp.sum(rank_incl * onehot, axis=1, keepdims=True)
    start_tok = jnp.sum(onehot * astart, axis=1, keepdims=True)
    pos_ref[...] = (start_tok + rank_tok - 1.0).astype(jnp.int32)

    ends = astart + pc
    total = jnp.sum(pc)
    jrow = lax.broadcasted_iota(jnp.int32, (_NB, 1), 0).astype(
        jnp.float32) * _T
    posj = jnp.minimum(jrow, total - _T)
    be = jnp.sum((ends <= posj).astype(jnp.int32), axis=1, keepdims=True)
    be_ref[...] = be
    bv_ref[...] = (jrow < total).astype(jnp.int32)


def _expert_body(be_ref, bv_ref, xs_ref, wg_ref, wu_ref, wd_ref, out_ref):
    j = pl.program_id(0)

    @pl.when(bv_ref[j] == 1)
    def _():
        xt = xs_ref[...].T
        g = jnp.dot(wg_ref[0], xt, preferred_element_type=jnp.float32)
        u = jnp.dot(wu_ref[0], xt, preferred_element_type=jnp.float32)
        h = (g / (1.0 + jnp.exp(-g))) * u
        ot = jnp.dot(wd_ref[0], h, preferred_element_type=jnp.float32)
        out_ref[...] = ot.T


_plan = pl.pallas_call(
    _plan_body,
    out_shape=(
        jax.ShapeDtypeStruct((_S, 1), jnp.int32),
        jax.ShapeDtypeStruct((_NB, 1), jnp.int32),
        jax.ShapeDtypeStruct((_NB, 1), jnp.int32),
    ),
)

_experts = pl.pallas_call(
    _expert_body,
    grid_spec=pltpu.PrefetchScalarGridSpec(
        num_scalar_prefetch=2,
        grid=(_NB,),
        in_specs=[
            pl.BlockSpec((_T, _H), lambda j, be, bv: (j, 0)),
            pl.BlockSpec((1, _FF, _H), lambda j, be, bv: (be[j], 0, 0)),
            pl.BlockSpec((1, _FF, _H), lambda j, be, bv: (be[j], 0, 0)),
            pl.BlockSpec((1, _H, _FF), lambda j, be, bv: (be[j], 0, 0)),
        ],
        out_specs=pl.BlockSpec((_T, _H), lambda j, be, bv: (j, 0)),
    ),
    out_shape=jax.ShapeDtypeStruct((_PS, _H), jnp.float32),
)

@functools.cache
def _sc_kernels():
    mesh = plsc.VectorSubcoreMesh(core_axis_name="c", subcore_axis_name="s")
    scratch = [
        pltpu.VMEM((_CHUNK,), jnp.int32),
        pltpu.VMEM((_CHUNK, _H), jnp.float32),
        pltpu.SemaphoreType.DMA,
    ]

    @functools.partial(
        pl.kernel,
        out_type=jax.ShapeDtypeStruct((_PS, _H), jnp.float32),
        mesh=mesh,
        scratch_types=scratch,
    )
    def dispatch(x_hbm, pos_hbm, xs_hbm, idx_v, rows_v, sem):
        wid = lax.axis_index("s") * 2 + lax.axis_index("c")
        base = wid * _CHUNK
        pltpu.sync_copy(pos_hbm.at[pl.ds(base, _CHUNK)], idx_v)
        pltpu.sync_copy(x_hbm.at[pl.ds(base, _CHUNK)], rows_v)
        pltpu.async_copy(rows_v, xs_hbm.at[idx_v], sem).wait()

    @functools.partial(
        pl.kernel,
        out_type=jax.ShapeDtypeStruct((_S, _H), jnp.float32),
        mesh=mesh,
        scratch_types=scratch,
    )
    def combine(po_hbm, pos_hbm, out_hbm, idx_v, rows_v, sem):
        wid = lax.axis_index("s") * 2 + lax.axis_index("c")
        base = wid * _CHUNK
        pltpu.sync_copy(pos_hbm.at[pl.ds(base, _CHUNK)], idx_v)
        pltpu.async_copy(po_hbm.at[idx_v], rows_v, sem).wait()
        pltpu.sync_copy(rows_v, out_hbm.at[pl.ds(base, _CHUNK)])

    return dispatch, combine


@jax.jit
def kernel(x, gate_w, w_gate, w_up, w_down):
    b, s, h = x.shape
    x_flat = x.reshape(s, h)
    dispatch, combine = _sc_kernels()
    pos2, be2, bv2 = _plan(x_flat, gate_w)
    pos = pos2.reshape(s)
    xs = dispatch(x_flat, pos)
    po = _experts(be2.reshape(-1), bv2.reshape(-1), xs, w_gate, w_up, w_down)
    out = combine(po, pos)
    return out.reshape(b, s, h)
